# baseline (speedup 1.0000x reference)
import functools

import jax
import jax.numpy as jnp
from jax.experimental import pallas as pl
from jax.experimental.pallas import tpu as pltpu


def _layer_body(apply_relu, first_layer, adj_ref, xblk_ref, fchunk_ref,
                wa_ref, wb_ref, deg_ref, out_ref, odeg_ref, p_scr, d_scr):
    k = pl.program_id(1)
    nk = pl.num_programs(1)
    a = adj_ref[...]
    pj = jnp.dot(a.astype(jnp.bfloat16), fchunk_ref[...],
                 preferred_element_type=jnp.float32)
    if first_layer:
        dj = jnp.sum(a, axis=1, keepdims=True)

    @pl.when(k == 0)
    def _():
        p_scr[...] = pj
        if first_layer:
            d_scr[...] = dj

    @pl.when(k > 0)
    def _():
        p_scr[...] = p_scr[...] + pj
        if first_layer:
            d_scr[...] = d_scr[...] + dj

    @pl.when(k == nk - 1)
    def _():
        if first_layer:
            deg = d_scr[...] + 1.0
        else:
            deg = deg_ref[...]
        odeg_ref[...] = deg
        neigh = (p_scr[...] / deg).astype(jnp.bfloat16)
        out = (jnp.dot(xblk_ref[...], wa_ref[...],
                       preferred_element_type=jnp.float32)
               + jnp.dot(neigh, wb_ref[...],
                         preferred_element_type=jnp.float32))
        if apply_relu:
            out = jnp.maximum(out, 0.0)
        out_ref[...] = out.astype(out_ref.dtype)


def _sage_layer(adj, x16, wa, wb, deg, first_layer, apply_relu, out_dtype,
                bm, kc):
    n, d = x16.shape
    dh = wa.shape[1]
    out_shape = [jax.ShapeDtypeStruct((n, dh), out_dtype),
                 jax.ShapeDtypeStruct((n, 1), jnp.float32)]
    out_specs = [pl.BlockSpec((bm, dh), lambda i, k: (i, 0)),
                 pl.BlockSpec((bm, 1), lambda i, k: (i, 0))]
    res = pl.pallas_call(
        functools.partial(_layer_body, apply_relu, first_layer),
        grid=(n // bm, n // kc),
        in_specs=[
            pl.BlockSpec((bm, kc), lambda i, k: (i, k)),
            pl.BlockSpec((bm, d), lambda i, k: (i, 0)),
            pl.BlockSpec((kc, d), lambda i, k: (k, 0)),
            pl.BlockSpec((d, dh), lambda i, k: (0, 0)),
            pl.BlockSpec((d, dh), lambda i, k: (0, 0)),
            pl.BlockSpec((bm, 1), lambda i, k: (i, 0)),
        ],
        out_specs=out_specs,
        out_shape=out_shape,
        scratch_shapes=[
            pltpu.VMEM((bm, dh), jnp.float32),
            pltpu.VMEM((bm, 1), jnp.float32),
        ],
    )(adj, x16, x16, wa, wb, deg)
    return res


def kernel(adj, inputs, W0, W1):
    n, d_in = inputs.shape
    dh = W0.shape[0]
    x16 = inputs.astype(jnp.bfloat16)
    wa0, wb0 = (W0[:, :d_in].T.astype(jnp.bfloat16),
                W0[:, d_in:].T.astype(jnp.bfloat16))
    wa1, wb1 = (W1[:, :dh].T.astype(jnp.bfloat16),
                W1[:, dh:].T.astype(jnp.bfloat16))
    dummy_deg = jnp.ones((n, 1), jnp.float32)
    h, deg = _sage_layer(adj, x16, wa0, wb0, dummy_deg, first_layer=True,
                         apply_relu=True, out_dtype=jnp.bfloat16,
                         bm=512, kc=512)
    out, _ = _sage_layer(adj, h, wa1, wb1, deg, first_layer=False,
                         apply_relu=False, out_dtype=jnp.float32,
                         bm=512, kc=512)
    return out


# speedup vs baseline: 2.1293x; 2.1293x over previous
import functools

import jax
import jax.numpy as jnp
from jax.experimental import pallas as pl


def _layer_body(apply_relu, first_layer, bm, adj_ref, feats_ref, wa_ref,
                wb_ref, deg_ref, out_ref, odeg_ref):
    i = pl.program_id(0)
    a = adj_ref[...]
    p = jnp.dot(a.astype(jnp.bfloat16), feats_ref[...],
                preferred_element_type=jnp.float32)
    if first_layer:
        deg = jnp.sum(a, axis=1, keepdims=True) + 1.0
    else:
        deg = deg_ref[...]
    odeg_ref[...] = deg
    neigh = (p / deg).astype(jnp.bfloat16)
    xblk = feats_ref[pl.ds(i * bm, bm), :]
    out = (jnp.dot(xblk, wa_ref[...], preferred_element_type=jnp.float32)
           + jnp.dot(neigh, wb_ref[...], preferred_element_type=jnp.float32))
    if apply_relu:
        out = jnp.maximum(out, 0.0)
    out_ref[...] = out.astype(out_ref.dtype)


def _sage_layer(adj, x16, wa, wb, deg, first_layer, apply_relu, out_dtype, bm):
    n, d = x16.shape
    dh = wa.shape[1]
    return pl.pallas_call(
        functools.partial(_layer_body, apply_relu, first_layer, bm),
        grid=(n // bm,),
        in_specs=[
            pl.BlockSpec((bm, n), lambda i: (i, 0)),
            pl.BlockSpec((n, d), lambda i: (0, 0)),
            pl.BlockSpec((d, dh), lambda i: (0, 0)),
            pl.BlockSpec((d, dh), lambda i: (0, 0)),
            pl.BlockSpec((bm, 1), lambda i: (i, 0)),
        ],
        out_specs=[
            pl.BlockSpec((bm, dh), lambda i: (i, 0)),
            pl.BlockSpec((bm, 1), lambda i: (i, 0)),
        ],
        out_shape=[
            jax.ShapeDtypeStruct((n, dh), out_dtype),
            jax.ShapeDtypeStruct((n, 1), jnp.float32),
        ],
    )(adj, x16, wa, wb, deg)


def kernel(adj, inputs, W0, W1):
    n, d_in = inputs.shape
    dh = W0.shape[0]
    x16 = inputs.astype(jnp.bfloat16)
    wa0, wb0 = (W0[:, :d_in].T.astype(jnp.bfloat16),
                W0[:, d_in:].T.astype(jnp.bfloat16))
    wa1, wb1 = (W1[:, :dh].T.astype(jnp.bfloat16),
                W1[:, dh:].T.astype(jnp.bfloat16))
    dummy_deg = jnp.ones((n, 1), jnp.float32)
    h, deg = _sage_layer(adj, x16, wa0, wb0, dummy_deg, first_layer=True,
                         apply_relu=True, out_dtype=jnp.bfloat16, bm=512)
    out, _ = _sage_layer(adj, h, wa1, wb1, deg, first_layer=False,
                         apply_relu=False, out_dtype=jnp.float32, bm=512)
    return out


# speedup vs baseline: 2.2589x; 1.0609x over previous
import functools

import jax
import jax.numpy as jnp
from jax.experimental import pallas as pl


def _layer_body(apply_relu, first_layer, bm, adj_ref, xblk_ref, feats_ref,
                wa_ref, wb_ref, deg_ref, out_ref, odeg_ref):
    a = adj_ref[...]
    p = jnp.dot(a.astype(jnp.bfloat16), feats_ref[...],
                preferred_element_type=jnp.float32)
    if first_layer:
        deg = jnp.sum(a, axis=1, keepdims=True) + 1.0
    else:
        deg = deg_ref[...]
    odeg_ref[...] = deg
    neigh = (p / deg).astype(jnp.bfloat16)
    out = (jnp.dot(xblk_ref[...], wa_ref[...], preferred_element_type=jnp.float32)
           + jnp.dot(neigh, wb_ref[...], preferred_element_type=jnp.float32))
    if apply_relu:
        out = jnp.maximum(out, 0.0)
    out_ref[...] = out.astype(out_ref.dtype)


def _sage_layer(adj, x16, wa, wb, deg, first_layer, apply_relu, out_dtype, bm):
    n, d = x16.shape
    dh = wa.shape[1]
    return pl.pallas_call(
        functools.partial(_layer_body, apply_relu, first_layer, bm),
        grid=(n // bm,),
        in_specs=[
            pl.BlockSpec((bm, n), lambda i: (i, 0)),
            pl.BlockSpec((bm, d), lambda i: (i, 0)),
            pl.BlockSpec((n, d), lambda i: (0, 0)),
            pl.BlockSpec((d, dh), lambda i: (0, 0)),
            pl.BlockSpec((d, dh), lambda i: (0, 0)),
            pl.BlockSpec((bm, 1), lambda i: (i, 0)),
        ],
        out_specs=[
            pl.BlockSpec((bm, dh), lambda i: (i, 0)),
            pl.BlockSpec((bm, 1), lambda i: (i, 0)),
        ],
        out_shape=[
            jax.ShapeDtypeStruct((n, dh), out_dtype),
            jax.ShapeDtypeStruct((n, 1), jnp.float32),
        ],
    )(adj, x16, x16, wa, wb, deg)


def kernel(adj, inputs, W0, W1):
    n, d_in = inputs.shape
    dh = W0.shape[0]
    x16 = inputs.astype(jnp.bfloat16)
    wa0, wb0 = (W0[:, :d_in].T.astype(jnp.bfloat16),
                W0[:, d_in:].T.astype(jnp.bfloat16))
    wa1, wb1 = (W1[:, :dh].T.astype(jnp.bfloat16),
                W1[:, dh:].T.astype(jnp.bfloat16))
    dummy_deg = jnp.ones((n, 1), jnp.float32)
    h, deg = _sage_layer(adj, x16, wa0, wb0, dummy_deg, first_layer=True,
                         apply_relu=True, out_dtype=jnp.bfloat16, bm=512)
    out, _ = _sage_layer(adj, h, wa1, wb1, deg, first_layer=False,
                         apply_relu=False, out_dtype=jnp.float32, bm=512)
    return out
